# baseline (speedup 1.0000x reference)
import jax
import jax.numpy as jnp
from jax import lax
from jax.experimental import pallas as pl
from jax.experimental.pallas import tpu as pltpu
from jax.experimental.pallas import tpu_sc as plsc

EMB = 64
NC, NS = 2, 16
NW = NC * NS
NT = 200
BB = 128


def _emb_body(xv_hbm, table_hbm, out_hbm, idxT, rows_v, rowsT, gsem, ssem):
    wid = lax.axis_index("s") * NC + lax.axis_index("c")

    pltpu.sync_copy(xv_hbm.at[:, wid], idxT)

    e0 = lax.broadcasted_iota(jnp.int32, (16,), 0)
    eb_vecs = [(e0 + 16 * h) // 8 for h in range(4)]
    ei_vecs = [(e0 + 16 * h) % 8 for h in range(4)]

    def fire_gather(t, s):
        td = t // 8
        tm = t % 8
        pltpu.async_copy(
            table_hbm.at[idxT.at[td, tm]], rows_v.at[s], gsem
        )

    def wait_gather(s):
        pltpu.make_async_copy(
            table_hbm.at[pl.ds(0, BB)], rows_v.at[s], gsem
        ).wait()

    def wait_store(s):
        pltpu.make_async_copy(
            rowsT.at[s, :, :, pl.ds(0, BB)], out_hbm.at[0, :, wid], ssem
        ).wait()

    zero16 = e0 * 0

    def transpose(s):
        dst = rowsT.at[s % 2]

        @plsc.parallel_loop(0, BB, 1, unroll=8)
        def _bi(bi):
            bs = zero16 + bi
            for h in range(4):
                v = rows_v[s, bi, pl.ds(16 * h, 16)]
                plsc.store_scatter(dst, [eb_vecs[h], ei_vecs[h], bs], v)

    for s in range(7):
        fire_gather(s, s)
    pltpu.async_copy(rowsT.at[0, :, :, pl.ds(0, BB)], out_hbm.at[0, :, wid], ssem)
    pltpu.async_copy(rowsT.at[1, :, :, pl.ds(0, BB)], out_hbm.at[0, :, wid], ssem)

    @pl.loop(0, NT // 8)
    def _quad(tq):
        for s in range(8):
            t = 8 * tq + s
            wait_gather(s)
            fire_gather(jnp.minimum(t + 7, NT - 1), (s + 7) % 8)
            transpose(s)
            wait_store(s % 2)
            pltpu.async_copy(
                rowsT.at[s % 2, :, :, pl.ds(0, BB)],
                out_hbm.at[t, :, wid],
                ssem,
            )

    for s in range(2):
        wait_store(s)
    for s in range(7):
        wait_gather(s)


def kernel(x, W):
    xv = (
        x.astype(jnp.int32)
        .T.reshape(NT // 8, 8, NW, BB)
        .transpose(0, 2, 1, 3)
    )
    mesh = plsc.VectorSubcoreMesh(
        core_axis_name="c", subcore_axis_name="s",
        num_cores=NC, num_subcores=NS,
    )
    out5 = pl.kernel(
        _emb_body,
        out_type=jax.ShapeDtypeStruct((NT, 8, NW, 8, BB), jnp.float32),
        mesh=mesh,
        scratch_types=[
            pltpu.VMEM((NT // 8, 8, BB), jnp.int32),
            pltpu.VMEM((8, BB, EMB), jnp.float32),
            pltpu.VMEM((2, 8, 8, BB + 1), jnp.float32),
            pltpu.SemaphoreType.DMA,
            pltpu.SemaphoreType.DMA,
        ],
        compiler_params=pltpu.CompilerParams(
            use_tc_tiling_on_sc=False,
            needs_layout_passes=False,
            disable_bounds_checks=True,
        ),
    )(xv, W)
    return out5.transpose(2, 4, 0, 1, 3).reshape(x.shape[0], NT, EMB)


# speedup vs baseline: 1.0193x; 1.0193x over previous
import jax
import jax.numpy as jnp
from jax import lax
from jax.experimental import pallas as pl
from jax.experimental.pallas import tpu as pltpu
from jax.experimental.pallas import tpu_sc as plsc

EMB = 64
NC, NS = 2, 16
NW = NC * NS
NT = 200
BB = 128


def _emb_body(xv_hbm, table_hbm, out_hbm, idxT, rows_v, rowsT, gsem, ssem):
    wid = lax.axis_index("s") * NC + lax.axis_index("c")

    pltpu.sync_copy(xv_hbm.at[:, wid], idxT)

    e0 = lax.broadcasted_iota(jnp.int32, (16,), 0)
    eb_vecs = [(e0 + 16 * h) // 8 for h in range(4)]
    ei_vecs = [(e0 + 16 * h) % 8 for h in range(4)]

    def fire_gather(t, s):
        td = t // 8
        tm = t % 8
        pltpu.async_copy(
            table_hbm.at[idxT.at[td, tm]], rows_v.at[s], gsem
        )

    def wait_gather(s):
        pltpu.make_async_copy(
            table_hbm.at[pl.ds(0, BB)], rows_v.at[s], gsem
        ).wait()

    def wait_store(s):
        pltpu.make_async_copy(
            rowsT.at[s, :, :, pl.ds(0, BB)], out_hbm.at[0, :, wid], ssem
        ).wait()

    zero16 = e0 * 0

    def transpose(s):
        dst = rowsT.at[s % 2]

        @plsc.parallel_loop(0, BB, 1, unroll=8)
        def _bi(bi):
            bs = zero16 + bi
            for h in range(4):
                v = rows_v[s, bi, pl.ds(16 * h, 16)]
                plsc.store_scatter(dst, [eb_vecs[h], ei_vecs[h], bs], v)

    for s in range(3):
        fire_gather(s, s)
    pltpu.async_copy(rowsT.at[0, :, :, pl.ds(0, BB)], out_hbm.at[0, :, wid], ssem)
    pltpu.async_copy(rowsT.at[1, :, :, pl.ds(0, BB)], out_hbm.at[0, :, wid], ssem)

    @pl.loop(0, NT // 4)
    def _quad(tq):
        for s in range(4):
            t = 4 * tq + s
            wait_gather(s)
            fire_gather(jnp.minimum(t + 3, NT - 1), (s + 3) % 4)
            transpose(s)
            wait_store(s % 2)
            pltpu.async_copy(
                rowsT.at[s % 2, :, :, pl.ds(0, BB)],
                out_hbm.at[t, :, wid],
                ssem,
            )

    for s in range(2):
        wait_store(s)
    for s in range(3):
        wait_gather(s)


def kernel(x, W):
    xv = (
        x.astype(jnp.int32)
        .T.reshape(NT // 8, 8, NW, BB)
        .transpose(0, 2, 1, 3)
    )
    mesh = plsc.VectorSubcoreMesh(
        core_axis_name="c", subcore_axis_name="s",
        num_cores=NC, num_subcores=NS,
    )
    out5 = pl.kernel(
        _emb_body,
        out_type=jax.ShapeDtypeStruct((NT, 8, NW, 8, BB), jnp.float32),
        mesh=mesh,
        scratch_types=[
            pltpu.VMEM((NT // 8, 8, BB), jnp.int32),
            pltpu.VMEM((4, BB, EMB), jnp.float32),
            pltpu.VMEM((2, 8, 8, BB + 1), jnp.float32),
            pltpu.SemaphoreType.DMA,
            pltpu.SemaphoreType.DMA,
        ],
        compiler_params=pltpu.CompilerParams(
            use_tc_tiling_on_sc=False,
            needs_layout_passes=False,
            disable_bounds_checks=True,
        ),
    )(xv, W)
    return out5.transpose(2, 4, 0, 1, 3).reshape(x.shape[0], NT, EMB)
